# baseline (speedup 1.0000x reference)
import dataclasses
import jax
import jax.numpy as jnp
from jax import lax
from jax.experimental import pallas as pl
from jax.experimental.pallas import tpu as pltpu
from jax.experimental.pallas import tpu_sc as plsc

_N = 10000
_E = 320000
_NT = 16
_NPAD = 10240
_S = _NPAD // _NT
_EC = _E // _NT
_MAGIC = 0x5F3759DF


def _matvec_body(x_ref, w_ref, o_ref):
    o_ref[pl.ds(0, _N), :] = jnp.dot(x_ref[...], w_ref[...],
                                     preferred_element_type=jnp.float32)
    o_ref[pl.ds(_N, _NPAD - _N), :] = jnp.zeros((_NPAD - _N, 1), jnp.float32)


def _sc_gcn(z, packed_edges, params):
    mesh = plsc.VectorSubcoreMesh(core_axis_name="c", subcore_axis_name="s")
    cp = pltpu.CompilerParams()
    if "needs_layout_passes" in pltpu.CompilerParams.__dataclass_fields__:
        cp = dataclasses.replace(cp, needs_layout_passes=False)

    vec = jax.ShapeDtypeStruct((_NPAD,), jnp.float32)
    out_type = [vec, vec, vec, vec, jax.ShapeDtypeStruct((16,), jnp.float32)]

    @pl.kernel(
        mesh=mesh, out_type=out_type, compiler_params=cp,
        scratch_types=[
            pltpu.VMEM((_EC,), jnp.int32),
            pltpu.VMEM((_NPAD,), jnp.float32),
            pltpu.VMEM((_NPAD,), jnp.float32),
            pltpu.VMEM((_S,), jnp.float32),
            pltpu.VMEM((_S,), jnp.float32),
            pltpu.VMEM((_S,), jnp.float32),
            pltpu.VMEM((_S,), jnp.float32),
            pltpu.VMEM((16,), jnp.float32),
            pltpu.VMEM((64,), jnp.float32),
            pltpu.SemaphoreType.DMA,
            pltpu.VMEM_SHARED((_NPAD,), jnp.float32),
            pltpu.VMEM_SHARED((_NT, _NPAD), jnp.float32),
        ])
    def k(z_hbm, edge_hbm, par_hbm,
          y1_hbm, y2_hbm, y3_hbm, y4_hbm, g_hbm,
          edge_v, u_loc, out_loc, dinv_v, v_loc, tmp_v, part_v,
          par_v, msum64_v, dma_sem, u_sh, parts_sh):
        cid = lax.axis_index("c")
        t = lax.axis_index("s")

        @pl.when(cid == 0)
        def _():
            base_e = t * _EC
            base_n = t * _S
            zeros16 = jnp.zeros((16,), jnp.float32)
            ones16 = jnp.ones((16,), jnp.float32)

            pltpu.sync_copy(par_hbm, par_v)
            pltpu.sync_copy(edge_hbm.at[pl.ds(base_e, _EC)], edge_v)

            @pl.loop(0, _NPAD, step=64)
            def _(i):
                for q in range(4):
                    out_loc[pl.ds(i + 16 * q, 16)] = zeros16

            @pl.loop(0, _EC, step=16)
            def _(j):
                dv = edge_v[pl.ds(j, 16)] & 0x3FFF
                plsc.addupdate_scatter(out_loc, [dv], ones16)

            pltpu.sync_copy(out_loc, parts_sh.at[t])
            plsc.subcore_barrier()

            @pl.loop(0, _S, step=16)
            def _(i):
                tmp_v[pl.ds(i, 16)] = ones16
            for p in range(_NT):
                pltpu.sync_copy(parts_sh.at[p, pl.ds(base_n, _S)], part_v)

                @pl.loop(0, _S, step=64)
                def _(i):
                    for q in range(4):
                        tmp_v[pl.ds(i + 16 * q, 16)] = (
                            tmp_v[pl.ds(i + 16 * q, 16)]
                            + part_v[pl.ds(i + 16 * q, 16)])

            @pl.loop(0, _S, step=16)
            def _(i):
                d = tmp_v[pl.ds(i, 16)]
                yi = _MAGIC - lax.shift_right_logical(
                    lax.bitcast_convert_type(d, jnp.int32), 1)
                y = lax.bitcast_convert_type(yi, jnp.float32)
                y = y * (1.5 - 0.5 * d * y * y)
                y = y * (1.5 - 0.5 * d * y * y)
                y = y * (1.5 - 0.5 * d * y * y)
                dinv_v[pl.ds(i, 16)] = y

            pltpu.sync_copy(z_hbm.at[pl.ds(base_n, _S)], v_loc)

            y_hbms = [y1_hbm, y2_hbm, y3_hbm, y4_hbm]
            for kk in range(4):
                pv = par_v[...]
                w_s = pv[2 * kk]
                b_s = pv[2 * kk + 1]

                @pl.loop(0, _S, step=16)
                def _(i):
                    tmp_v[pl.ds(i, 16)] = (dinv_v[pl.ds(i, 16)]
                                           * v_loc[pl.ds(i, 16)])
                pltpu.sync_copy(tmp_v, u_sh.at[pl.ds(base_n, _S)])
                plsc.subcore_barrier()

                u_cp = pltpu.async_copy(u_sh, u_loc, dma_sem)

                @pl.loop(0, _NPAD, step=64)
                def _(i):
                    for q in range(4):
                        out_loc[pl.ds(i + 16 * q, 16)] = zeros16

                u_cp.wait()

                @pl.loop(0, _EC, step=16)
                def _(j):
                    ev = edge_v[pl.ds(j, 16)]
                    g = plsc.load_gather(
                        u_loc, [lax.shift_right_logical(ev, 14)])
                    plsc.addupdate_scatter(out_loc, [ev & 0x3FFF], g)

                pltpu.sync_copy(out_loc, parts_sh.at[t])
                plsc.subcore_barrier()

                for p in range(_NT):
                    pltpu.sync_copy(parts_sh.at[p, pl.ds(base_n, _S)], part_v)

                    @pl.loop(0, _S, step=64)
                    def _(i):
                        for q in range(4):
                            tmp_v[pl.ds(i + 16 * q, 16)] = (
                                tmp_v[pl.ds(i + 16 * q, 16)]
                                + part_v[pl.ds(i + 16 * q, 16)])

                def _vnext_body(i2, ps, w_s=w_s, b_s=b_s):
                    i = i2 * 16
                    vn = (w_s * (dinv_v[pl.ds(i, 16)]
                                 * tmp_v[pl.ds(i, 16)]) + b_s)
                    v_loc[pl.ds(i, 16)] = vn
                    keep = (base_n + i) < _N
                    return ps + jnp.where(keep, vn, 0.0)

                msum64_v[pl.ds(16 * kk, 16)] = lax.fori_loop(
                    0, _S // 16, _vnext_body, zeros16)
                pltpu.sync_copy(v_loc, y_hbms[kk].at[pl.ds(base_n, _S)])

            pltpu.sync_copy(msum64_v, parts_sh.at[t, pl.ds(0, 64)])
            plsc.subcore_barrier()

            @pl.when(t == 0)
            def _():
                lanes = lax.iota(jnp.int32, 16)
                gv = zeros16
                accs = [zeros16] * 4
                for p in range(_NT):
                    pltpu.sync_copy(parts_sh.at[p, pl.ds(0, 64)], msum64_v)
                    for kk in range(4):
                        accs[kk] = accs[kk] + msum64_v[pl.ds(16 * kk, 16)]
                for kk in range(4):
                    s = jnp.sum(accs[kk]) * jnp.float32(1.0 / _N)
                    gv = jnp.where(lanes == kk, s, gv)
                par_v[...] = gv
                pltpu.sync_copy(par_v, g_hbm)

    return k(z, packed_edges, params)


def kernel(x, edge_index, W1, b1, W2, b2, W3, b3, W4, b4):
    z = pl.pallas_call(
        _matvec_body,
        out_shape=jax.ShapeDtypeStruct((_NPAD, 1), jnp.float32),
    )(x, W1)
    params = jnp.concatenate([
        jnp.ones((1,), jnp.float32), b1, W2[0], b2, W3[0], b3, W4[0], b4,
        jnp.zeros((8,), jnp.float32)])
    packed = edge_index[0] * 16384 + edge_index[1]
    y1, y2, y3, y4, gvec = _sc_gcn(z[:, 0], packed, params)
    x_node = jnp.stack([y1[:_N], y2[:_N], y3[:_N], y4[:_N]], axis=1)
    return (gvec[:4], x_node)


# speedup vs baseline: 1.0470x; 1.0470x over previous
import dataclasses
import jax
import jax.numpy as jnp
from jax import lax
from jax.experimental import pallas as pl
from jax.experimental.pallas import tpu as pltpu
from jax.experimental.pallas import tpu_sc as plsc

_N = 10000
_E = 320000
_NT = 16
_NPAD = 10240
_S = _NPAD // _NT
_EC = _E // _NT
_MAGIC = 0x5F3759DF


def _matvec_body(x_ref, w_ref, o_ref):
    o_ref[pl.ds(0, _N), :] = jnp.dot(x_ref[...], w_ref[...],
                                     preferred_element_type=jnp.float32)
    o_ref[pl.ds(_N, _NPAD - _N), :] = jnp.zeros((_NPAD - _N, 1), jnp.float32)


def _sc_gcn(z, srcs, dsts, params):
    mesh = plsc.VectorSubcoreMesh(core_axis_name="c", subcore_axis_name="s")
    cp = pltpu.CompilerParams()
    if "needs_layout_passes" in pltpu.CompilerParams.__dataclass_fields__:
        cp = dataclasses.replace(cp, needs_layout_passes=False)

    vec = jax.ShapeDtypeStruct((_NPAD,), jnp.float32)
    out_type = [vec, vec, vec, vec, jax.ShapeDtypeStruct((16,), jnp.float32)]

    @pl.kernel(
        mesh=mesh, out_type=out_type, compiler_params=cp,
        scratch_types=[
            pltpu.VMEM((_EC,), jnp.int32),
            pltpu.VMEM((_EC,), jnp.int32),
            pltpu.VMEM((_NPAD,), jnp.float32),
            pltpu.VMEM((_NPAD,), jnp.float32),
            pltpu.VMEM((_S,), jnp.float32),
            pltpu.VMEM((_S,), jnp.float32),
            pltpu.VMEM((_S,), jnp.float32),
            pltpu.VMEM((_S,), jnp.float32),
            pltpu.VMEM((16,), jnp.float32),
            pltpu.VMEM((64,), jnp.float32),
            pltpu.SemaphoreType.DMA,
            pltpu.VMEM_SHARED((_NPAD,), jnp.float32),
            pltpu.VMEM_SHARED((_NT, _NPAD), jnp.float32),
        ])
    def k(z_hbm, src_hbm, dst_hbm, par_hbm,
          y1_hbm, y2_hbm, y3_hbm, y4_hbm, g_hbm,
          src_v, dst_v, u_loc, out_loc, dinv_v, v_loc, tmp_v, part_v,
          par_v, msum64_v, dma_sem, u_sh, parts_sh):
        cid = lax.axis_index("c")
        t = lax.axis_index("s")

        @pl.when(cid == 0)
        def _():
            base_e = t * _EC
            base_n = t * _S
            zeros16 = jnp.zeros((16,), jnp.float32)
            ones16 = jnp.ones((16,), jnp.float32)

            pltpu.sync_copy(par_hbm, par_v)
            pltpu.sync_copy(src_hbm.at[pl.ds(base_e, _EC)], src_v)
            pltpu.sync_copy(dst_hbm.at[pl.ds(base_e, _EC)], dst_v)

            @pl.loop(0, _NPAD, step=64)
            def _(i):
                for q in range(4):
                    out_loc[pl.ds(i + 16 * q, 16)] = zeros16

            @pl.loop(0, _EC, step=16)
            def _(j):
                plsc.addupdate_scatter(out_loc, [dst_v[pl.ds(j, 16)]], ones16)

            pltpu.sync_copy(out_loc, parts_sh.at[t])
            plsc.subcore_barrier()

            @pl.loop(0, _S, step=16)
            def _(i):
                tmp_v[pl.ds(i, 16)] = ones16
            for p in range(_NT):
                pltpu.sync_copy(parts_sh.at[p, pl.ds(base_n, _S)], part_v)

                @pl.loop(0, _S, step=64)
                def _(i):
                    for q in range(4):
                        tmp_v[pl.ds(i + 16 * q, 16)] = (
                            tmp_v[pl.ds(i + 16 * q, 16)]
                            + part_v[pl.ds(i + 16 * q, 16)])

            @pl.loop(0, _S, step=16)
            def _(i):
                d = tmp_v[pl.ds(i, 16)]
                yi = _MAGIC - lax.shift_right_logical(
                    lax.bitcast_convert_type(d, jnp.int32), 1)
                y = lax.bitcast_convert_type(yi, jnp.float32)
                y = y * (1.5 - 0.5 * d * y * y)
                y = y * (1.5 - 0.5 * d * y * y)
                y = y * (1.5 - 0.5 * d * y * y)
                dinv_v[pl.ds(i, 16)] = y

            pltpu.sync_copy(z_hbm.at[pl.ds(base_n, _S)], v_loc)

            y_hbms = [y1_hbm, y2_hbm, y3_hbm, y4_hbm]
            for kk in range(4):
                pv = par_v[...]
                w_s = pv[2 * kk]
                b_s = pv[2 * kk + 1]

                @pl.loop(0, _S, step=16)
                def _(i):
                    tmp_v[pl.ds(i, 16)] = (dinv_v[pl.ds(i, 16)]
                                           * v_loc[pl.ds(i, 16)])
                pltpu.sync_copy(tmp_v, u_sh.at[pl.ds(base_n, _S)])
                plsc.subcore_barrier()

                u_cp = pltpu.async_copy(u_sh, u_loc, dma_sem)

                @pl.loop(0, _NPAD, step=64)
                def _(i):
                    for q in range(4):
                        out_loc[pl.ds(i + 16 * q, 16)] = zeros16

                u_cp.wait()

                @pl.loop(0, _EC, step=16)
                def _(j):
                    g = plsc.load_gather(u_loc, [src_v[pl.ds(j, 16)]])
                    plsc.addupdate_scatter(out_loc, [dst_v[pl.ds(j, 16)]], g)

                pltpu.sync_copy(out_loc, parts_sh.at[t])
                plsc.subcore_barrier()

                for p in range(_NT):
                    pltpu.sync_copy(parts_sh.at[p, pl.ds(base_n, _S)], part_v)

                    @pl.loop(0, _S, step=64)
                    def _(i):
                        for q in range(4):
                            tmp_v[pl.ds(i + 16 * q, 16)] = (
                                tmp_v[pl.ds(i + 16 * q, 16)]
                                + part_v[pl.ds(i + 16 * q, 16)])

                def _vnext_body(i2, ps, w_s=w_s, b_s=b_s):
                    i = i2 * 16
                    vn = (w_s * (dinv_v[pl.ds(i, 16)]
                                 * tmp_v[pl.ds(i, 16)]) + b_s)
                    v_loc[pl.ds(i, 16)] = vn
                    keep = (base_n + i) < _N
                    return ps + jnp.where(keep, vn, 0.0)

                msum64_v[pl.ds(16 * kk, 16)] = lax.fori_loop(
                    0, _S // 16, _vnext_body, zeros16)
                pltpu.sync_copy(v_loc, y_hbms[kk].at[pl.ds(base_n, _S)])

            pltpu.sync_copy(msum64_v, parts_sh.at[t, pl.ds(0, 64)])
            plsc.subcore_barrier()

            @pl.when(t == 0)
            def _():
                lanes = lax.iota(jnp.int32, 16)
                gv = zeros16
                accs = [zeros16] * 4
                for p in range(_NT):
                    pltpu.sync_copy(parts_sh.at[p, pl.ds(0, 64)], msum64_v)
                    for kk in range(4):
                        accs[kk] = accs[kk] + msum64_v[pl.ds(16 * kk, 16)]
                for kk in range(4):
                    s = jnp.sum(accs[kk]) * jnp.float32(1.0 / _N)
                    gv = jnp.where(lanes == kk, s, gv)
                par_v[...] = gv
                pltpu.sync_copy(par_v, g_hbm)

    return k(z, srcs, dsts, params)


def kernel(x, edge_index, W1, b1, W2, b2, W3, b3, W4, b4):
    z = pl.pallas_call(
        _matvec_body,
        out_shape=jax.ShapeDtypeStruct((_NPAD, 1), jnp.float32),
    )(x, W1)
    params = jnp.concatenate([
        jnp.ones((1,), jnp.float32), b1, W2[0], b2, W3[0], b3, W4[0], b4,
        jnp.zeros((8,), jnp.float32)])
    y1, y2, y3, y4, gvec = _sc_gcn(z[:, 0], edge_index[0], edge_index[1],
                                   params)
    x_node = jnp.stack([y1[:_N], y2[:_N], y3[:_N], y4[:_N]], axis=1)
    return (gvec[:4], x_node)


# speedup vs baseline: 1.0593x; 1.0117x over previous
import dataclasses
import jax
import jax.numpy as jnp
from jax import lax
from jax.experimental import pallas as pl
from jax.experimental.pallas import tpu as pltpu
from jax.experimental.pallas import tpu_sc as plsc

_N = 10000
_E = 320000
_NT = 16
_NPAD = 10240
_S = _NPAD // _NT
_EC = _E // _NT
_MAGIC = 0x5F3759DF


def _matvec_body(x_ref, w_ref, o_ref):
    o_ref[pl.ds(0, _N), :] = jnp.dot(x_ref[...], w_ref[...],
                                     preferred_element_type=jnp.float32)
    o_ref[pl.ds(_N, _NPAD - _N), :] = jnp.zeros((_NPAD - _N, 1), jnp.float32)


def _sc_gcn(z, srcs, dsts, params):
    mesh = plsc.VectorSubcoreMesh(core_axis_name="c", subcore_axis_name="s",
                                  num_cores=1)
    cp = pltpu.CompilerParams()
    if "needs_layout_passes" in pltpu.CompilerParams.__dataclass_fields__:
        cp = dataclasses.replace(cp, needs_layout_passes=False)

    vec = jax.ShapeDtypeStruct((_NPAD,), jnp.float32)
    out_type = [vec, vec, vec, vec, jax.ShapeDtypeStruct((16,), jnp.float32)]

    @pl.kernel(
        mesh=mesh, out_type=out_type, compiler_params=cp,
        scratch_types=[
            pltpu.VMEM((_EC,), jnp.int32),
            pltpu.VMEM((_EC,), jnp.int32),
            pltpu.VMEM((_NPAD,), jnp.float32),
            pltpu.VMEM((_NPAD,), jnp.float32),
            pltpu.VMEM((_S,), jnp.float32),
            pltpu.VMEM((_S,), jnp.float32),
            pltpu.VMEM((_S,), jnp.float32),
            pltpu.VMEM((_S,), jnp.float32),
            pltpu.VMEM((16,), jnp.float32),
            pltpu.VMEM((64,), jnp.float32),
            pltpu.SemaphoreType.DMA,
            pltpu.VMEM_SHARED((_NPAD,), jnp.float32),
            pltpu.VMEM_SHARED((_NT, _NPAD), jnp.float32),
        ])
    def k(z_hbm, src_hbm, dst_hbm, par_hbm,
          y1_hbm, y2_hbm, y3_hbm, y4_hbm, g_hbm,
          src_v, dst_v, u_loc, out_loc, dinv_v, v_loc, tmp_v, part_v,
          par_v, msum64_v, dma_sem, u_sh, parts_sh):
        cid = lax.axis_index("c")
        t = lax.axis_index("s")

        @pl.when(cid == 0)
        def _():
            base_e = t * _EC
            base_n = t * _S
            zeros16 = jnp.zeros((16,), jnp.float32)
            ones16 = jnp.ones((16,), jnp.float32)

            pltpu.sync_copy(par_hbm, par_v)
            pltpu.sync_copy(src_hbm.at[pl.ds(base_e, _EC)], src_v)
            pltpu.sync_copy(dst_hbm.at[pl.ds(base_e, _EC)], dst_v)

            @pl.loop(0, _NPAD, step=64)
            def _(i):
                for q in range(4):
                    out_loc[pl.ds(i + 16 * q, 16)] = zeros16

            @pl.loop(0, _EC, step=16)
            def _(j):
                plsc.addupdate_scatter(out_loc, [dst_v[pl.ds(j, 16)]], ones16)

            pltpu.sync_copy(out_loc, parts_sh.at[t])
            plsc.subcore_barrier()

            @pl.loop(0, _S, step=16)
            def _(i):
                tmp_v[pl.ds(i, 16)] = ones16
            for p in range(_NT):
                pltpu.sync_copy(parts_sh.at[p, pl.ds(base_n, _S)], part_v)

                @pl.loop(0, _S, step=64)
                def _(i):
                    for q in range(4):
                        tmp_v[pl.ds(i + 16 * q, 16)] = (
                            tmp_v[pl.ds(i + 16 * q, 16)]
                            + part_v[pl.ds(i + 16 * q, 16)])

            @pl.loop(0, _S, step=16)
            def _(i):
                d = tmp_v[pl.ds(i, 16)]
                yi = _MAGIC - lax.shift_right_logical(
                    lax.bitcast_convert_type(d, jnp.int32), 1)
                y = lax.bitcast_convert_type(yi, jnp.float32)
                y = y * (1.5 - 0.5 * d * y * y)
                y = y * (1.5 - 0.5 * d * y * y)
                y = y * (1.5 - 0.5 * d * y * y)
                dinv_v[pl.ds(i, 16)] = y

            pltpu.sync_copy(z_hbm.at[pl.ds(base_n, _S)], v_loc)

            y_hbms = [y1_hbm, y2_hbm, y3_hbm, y4_hbm]
            for kk in range(4):
                pv = par_v[...]
                w_s = pv[2 * kk]
                b_s = pv[2 * kk + 1]

                @pl.loop(0, _S, step=16)
                def _(i):
                    tmp_v[pl.ds(i, 16)] = (dinv_v[pl.ds(i, 16)]
                                           * v_loc[pl.ds(i, 16)])
                pltpu.sync_copy(tmp_v, u_sh.at[pl.ds(base_n, _S)])
                plsc.subcore_barrier()

                u_cp = pltpu.async_copy(u_sh, u_loc, dma_sem)

                @pl.loop(0, _NPAD, step=64)
                def _(i):
                    for q in range(4):
                        out_loc[pl.ds(i + 16 * q, 16)] = zeros16

                u_cp.wait()

                @pl.loop(0, _EC, step=16)
                def _(j):
                    g = plsc.load_gather(u_loc, [src_v[pl.ds(j, 16)]])
                    plsc.addupdate_scatter(out_loc, [dst_v[pl.ds(j, 16)]], g)

                pltpu.sync_copy(out_loc, parts_sh.at[t])
                plsc.subcore_barrier()

                for p in range(_NT):
                    pltpu.sync_copy(parts_sh.at[p, pl.ds(base_n, _S)], part_v)

                    @pl.loop(0, _S, step=64)
                    def _(i):
                        for q in range(4):
                            tmp_v[pl.ds(i + 16 * q, 16)] = (
                                tmp_v[pl.ds(i + 16 * q, 16)]
                                + part_v[pl.ds(i + 16 * q, 16)])

                def _vnext_body(i2, ps, w_s=w_s, b_s=b_s):
                    i = i2 * 16
                    vn = (w_s * (dinv_v[pl.ds(i, 16)]
                                 * tmp_v[pl.ds(i, 16)]) + b_s)
                    v_loc[pl.ds(i, 16)] = vn
                    keep = (base_n + i) < _N
                    return ps + jnp.where(keep, vn, 0.0)

                msum64_v[pl.ds(16 * kk, 16)] = lax.fori_loop(
                    0, _S // 16, _vnext_body, zeros16)
                pltpu.sync_copy(v_loc, y_hbms[kk].at[pl.ds(base_n, _S)])

            pltpu.sync_copy(msum64_v, parts_sh.at[t, pl.ds(0, 64)])
            plsc.subcore_barrier()

            @pl.when(t == 0)
            def _():
                lanes = lax.iota(jnp.int32, 16)
                gv = zeros16
                accs = [zeros16] * 4
                for p in range(_NT):
                    pltpu.sync_copy(parts_sh.at[p, pl.ds(0, 64)], msum64_v)
                    for kk in range(4):
                        accs[kk] = accs[kk] + msum64_v[pl.ds(16 * kk, 16)]
                for kk in range(4):
                    s = jnp.sum(accs[kk]) * jnp.float32(1.0 / _N)
                    gv = jnp.where(lanes == kk, s, gv)
                par_v[...] = gv
                pltpu.sync_copy(par_v, g_hbm)

    return k(z, srcs, dsts, params)


def kernel(x, edge_index, W1, b1, W2, b2, W3, b3, W4, b4):
    z = pl.pallas_call(
        _matvec_body,
        out_shape=jax.ShapeDtypeStruct((_NPAD, 1), jnp.float32),
    )(x, W1)
    params = jnp.concatenate([
        jnp.ones((1,), jnp.float32), b1, W2[0], b2, W3[0], b3, W4[0], b4,
        jnp.zeros((8,), jnp.float32)])
    y1, y2, y3, y4, gvec = _sc_gcn(z[:, 0], edge_index[0], edge_index[1],
                                   params)
    x_node = jnp.stack([y1[:_N], y2[:_N], y3[:_N], y4[:_N]], axis=1)
    return (gvec[:4], x_node)


# speedup vs baseline: 1.0601x; 1.0007x over previous
import dataclasses
import jax
import jax.numpy as jnp
from jax import lax
from jax.experimental import pallas as pl
from jax.experimental.pallas import tpu as pltpu
from jax.experimental.pallas import tpu_sc as plsc

_N = 10000
_E = 320000
_NT = 16
_NPAD = 10240
_S = _NPAD // _NT
_EC = _E // _NT
_MAGIC = 0x5F3759DF


def _matvec_body(x_ref, w_ref, o_ref):
    o_ref[pl.ds(0, _N), :] = jnp.dot(x_ref[...], w_ref[...],
                                     preferred_element_type=jnp.float32)
    o_ref[pl.ds(_N, _NPAD - _N), :] = jnp.zeros((_NPAD - _N, 1), jnp.float32)


def _sc_gcn(z, srcs, dsts, params):
    mesh = plsc.VectorSubcoreMesh(core_axis_name="c", subcore_axis_name="s",
                                  num_cores=1)
    cp = pltpu.CompilerParams()
    if "needs_layout_passes" in pltpu.CompilerParams.__dataclass_fields__:
        cp = dataclasses.replace(cp, needs_layout_passes=False)

    vec = jax.ShapeDtypeStruct((_NPAD,), jnp.float32)
    out_type = [vec, vec, vec, vec, jax.ShapeDtypeStruct((16,), jnp.float32)]

    @pl.kernel(
        mesh=mesh, out_type=out_type, compiler_params=cp,
        scratch_types=[
            pltpu.VMEM((_EC,), jnp.int32),
            pltpu.VMEM((_EC,), jnp.int32),
            pltpu.VMEM((_NPAD,), jnp.float32),
            pltpu.VMEM((_NPAD,), jnp.float32),
            pltpu.VMEM((_S,), jnp.float32),
            pltpu.VMEM((_S,), jnp.float32),
            pltpu.VMEM((_S,), jnp.float32),
            pltpu.VMEM((_S,), jnp.float32),
            pltpu.VMEM((16,), jnp.float32),
            pltpu.VMEM((64,), jnp.float32),
            pltpu.SemaphoreType.DMA,
            pltpu.VMEM_SHARED((_NPAD,), jnp.float32),
            pltpu.VMEM_SHARED((_NT, _NPAD), jnp.float32),
        ])
    def k(z_hbm, src_hbm, dst_hbm, par_hbm,
          y1_hbm, y2_hbm, y3_hbm, y4_hbm, g_hbm,
          src_v, dst_v, u_loc, out_loc, dinv_v, v_loc, tmp_v, part_v,
          par_v, msum64_v, dma_sem, u_sh, parts_sh):
        cid = lax.axis_index("c")
        t = lax.axis_index("s")

        @pl.when(cid == 0)
        def _():
            base_e = t * _EC
            base_n = t * _S
            zeros16 = jnp.zeros((16,), jnp.float32)
            ones16 = jnp.ones((16,), jnp.float32)

            def accum_partials():
                for p in range(_NT):
                    pltpu.sync_copy(parts_sh.at[p, pl.ds(base_n, _S)], part_v)

                    @pl.loop(0, _S, step=64)
                    def _(i):
                        for q in range(4):
                            tmp_v[pl.ds(i + 16 * q, 16)] = (
                                tmp_v[pl.ds(i + 16 * q, 16)]
                                + part_v[pl.ds(i + 16 * q, 16)])

            pltpu.sync_copy(par_hbm, par_v)
            pltpu.sync_copy(src_hbm.at[pl.ds(base_e, _EC)], src_v)
            pltpu.sync_copy(dst_hbm.at[pl.ds(base_e, _EC)], dst_v)

            @pl.loop(0, _NPAD, step=64)
            def _(i):
                for q in range(4):
                    out_loc[pl.ds(i + 16 * q, 16)] = zeros16

            @pl.loop(0, _EC, step=16)
            def _(j):
                plsc.addupdate_scatter(out_loc, [dst_v[pl.ds(j, 16)]], ones16)

            pltpu.sync_copy(out_loc, parts_sh.at[t])
            plsc.subcore_barrier()

            @pl.loop(0, _S, step=16)
            def _(i):
                tmp_v[pl.ds(i, 16)] = ones16
            accum_partials()

            @pl.loop(0, _S, step=16)
            def _(i):
                d = tmp_v[pl.ds(i, 16)]
                yi = _MAGIC - lax.shift_right_logical(
                    lax.bitcast_convert_type(d, jnp.int32), 1)
                y = lax.bitcast_convert_type(yi, jnp.float32)
                y = y * (1.5 - 0.5 * d * y * y)
                y = y * (1.5 - 0.5 * d * y * y)
                y = y * (1.5 - 0.5 * d * y * y)
                dinv_v[pl.ds(i, 16)] = y

            pltpu.sync_copy(z_hbm.at[pl.ds(base_n, _S)], v_loc)

            y_hbms = [y1_hbm, y2_hbm, y3_hbm, y4_hbm]
            for kk in range(4):
                pv = par_v[...]
                w_s = pv[2 * kk]
                b_s = pv[2 * kk + 1]

                @pl.loop(0, _S, step=16)
                def _(i):
                    tmp_v[pl.ds(i, 16)] = (dinv_v[pl.ds(i, 16)]
                                           * v_loc[pl.ds(i, 16)])
                pltpu.sync_copy(tmp_v, u_sh.at[pl.ds(base_n, _S)])
                plsc.subcore_barrier()

                u_cp = pltpu.async_copy(u_sh, u_loc, dma_sem)

                @pl.loop(0, _NPAD, step=64)
                def _(i):
                    for q in range(4):
                        out_loc[pl.ds(i + 16 * q, 16)] = zeros16

                u_cp.wait()

                @pl.loop(0, _EC, step=16)
                def _(j):
                    g = plsc.load_gather(u_loc, [src_v[pl.ds(j, 16)]])
                    plsc.addupdate_scatter(out_loc, [dst_v[pl.ds(j, 16)]], g)

                pltpu.sync_copy(out_loc, parts_sh.at[t])
                plsc.subcore_barrier()

                accum_partials()

                def _vnext_body(i2, ps, w_s=w_s, b_s=b_s):
                    i = i2 * 16
                    vn = (w_s * (dinv_v[pl.ds(i, 16)]
                                 * tmp_v[pl.ds(i, 16)]) + b_s)
                    v_loc[pl.ds(i, 16)] = vn
                    keep = (base_n + i) < _N
                    return ps + jnp.where(keep, vn, 0.0)

                msum64_v[pl.ds(16 * kk, 16)] = lax.fori_loop(
                    0, _S // 16, _vnext_body, zeros16)
                pltpu.sync_copy(v_loc, y_hbms[kk].at[pl.ds(base_n, _S)])

            pltpu.sync_copy(msum64_v, parts_sh.at[t, pl.ds(0, 64)])
            plsc.subcore_barrier()

            @pl.when(t == 0)
            def _():
                lanes = lax.iota(jnp.int32, 16)
                gv = zeros16
                accs = [zeros16] * 4
                for p in range(_NT):
                    pltpu.sync_copy(parts_sh.at[p, pl.ds(0, 64)], msum64_v)
                    for kk in range(4):
                        accs[kk] = accs[kk] + msum64_v[pl.ds(16 * kk, 16)]
                for kk in range(4):
                    s = jnp.sum(accs[kk]) * jnp.float32(1.0 / _N)
                    gv = jnp.where(lanes == kk, s, gv)
                par_v[...] = gv
                pltpu.sync_copy(par_v, g_hbm)

    return k(z, srcs, dsts, params)


def kernel(x, edge_index, W1, b1, W2, b2, W3, b3, W4, b4):
    z = pl.pallas_call(
        _matvec_body,
        out_shape=jax.ShapeDtypeStruct((_NPAD, 1), jnp.float32),
    )(x, W1)
    params = jnp.concatenate([
        jnp.ones((1,), jnp.float32), b1, W2[0], b2, W3[0], b3, W4[0], b4,
        jnp.zeros((8,), jnp.float32)])
    y1, y2, y3, y4, gvec = _sc_gcn(z[:, 0], edge_index[0], edge_index[1],
                                   params)
    x_node = jnp.stack([y1[:_N], y2[:_N], y3[:_N], y4[:_N]], axis=1)
    return (gvec[:4], x_node)
